# baseline (speedup 1.0000x reference)
import functools

import jax
import jax.numpy as jnp
from jax import lax
from jax.experimental import pallas as pl
from jax.experimental.pallas import tpu as pltpu
from jax.experimental.pallas import tpu_sc as plsc

N, C, S = 50000, 128, 8
BN = 5000
K = 400
NCHUNK = N // K
NW = 32
TRIPS = -(-NCHUNK // NW)


def _idx_kernel(f_ref, w_ref, b_ref, idx_ref):
    logits = jnp.dot(f_ref[...], w_ref[...], preferred_element_type=jnp.float32)
    logits = logits + b_ref[...]
    idx_ref[...] = jnp.argmax(logits, axis=1).astype(jnp.int32)[:, None]


def _route_idx(features, W, b):
    return pl.pallas_call(
        _idx_kernel,
        grid=(N // BN,),
        in_specs=[
            pl.BlockSpec((BN, C), lambda i: (i, 0)),
            pl.BlockSpec((C, S), lambda i: (0, 0)),
            pl.BlockSpec((S,), lambda i: (0,)),
        ],
        out_specs=pl.BlockSpec((BN, 1), lambda i: (i, 0)),
        out_shape=jax.ShapeDtypeStruct((N, 1), jnp.int32),
    )(features, W, b)


_mesh = plsc.VectorSubcoreMesh(core_axis_name="c", subcore_axis_name="s")


@functools.partial(
    pl.kernel,
    out_type=jax.ShapeDtypeStruct((S * N, C), jnp.float32),
    mesh=_mesh,
    scratch_types=[
        pltpu.VMEM((K,), jnp.int32),
        pltpu.VMEM((K, C), jnp.float32),
        pltpu.VMEM((K, C), jnp.float32),
    ]
    + [pltpu.VMEM((K,), jnp.int32) for _ in range(S + 1)]
    + [pltpu.SemaphoreType.DMA],
)
def _sc_scatter(f_hbm, e_hbm, z_hbm, out_hbm, e_v, rows_v, zero_v,
                t0, t1, t2, t3, t4, t5, t6, t7, t8, sem):
    trefs = (t0, t1, t2, t3, t4, t5, t6, t7, t8)
    wid = lax.axis_index("s") * 2 + lax.axis_index("c")
    pltpu.sync_copy(z_hbm, zero_v)

    def chunk_body(it, carry):
        chunk = wid + NW * it

        @pl.when(chunk < NCHUNK)
        def _():
            base = chunk * K
            pltpu.sync_copy(f_hbm.at[pl.ds(base, K)], rows_v)
            pltpu.sync_copy(e_hbm.at[pl.ds(base, K)], e_v)
            nsplat = jnp.full((16,), N, jnp.int32)
            for g in range(K // 16):
                e16 = e_v[pl.ds(g * 16, 16)]
                r16 = jnp.full((16,), base + g * 16, jnp.int32) + lax.iota(
                    jnp.int32, 16
                )
                for s in range(S):
                    alt = (s + 1) % S
                    slab = jnp.where(
                        e16 == jnp.full((16,), s, jnp.int32),
                        jnp.full((16,), alt, jnp.int32),
                        jnp.full((16,), s, jnp.int32),
                    )
                    trefs[s][pl.ds(g * 16, 16)] = slab * nsplat + r16
                trefs[S][pl.ds(g * 16, 16)] = e16 * nsplat + r16
            copies = [
                pltpu.async_copy(zero_v, out_hbm.at[trefs[s]], sem)
                for s in range(S)
            ]
            copies.append(pltpu.async_copy(rows_v, out_hbm.at[trefs[S]], sem))
            for cp in copies:
                cp.wait()

        return carry

    lax.fori_loop(0, TRIPS, chunk_body, 0)


@functools.partial(jax.jit, static_argnames=())
def kernel(features, W, b):
    idx = _route_idx(features, W, b).reshape(N)
    zeros_rows = jnp.zeros((K, C), jnp.float32)
    flat = _sc_scatter(features, idx, zeros_rows)
    return flat.reshape(S, N, C)


# speedup vs baseline: 1.4703x; 1.4703x over previous
import functools

import jax
import jax.numpy as jnp
from jax.experimental import pallas as pl
from jax.experimental.pallas import tpu as pltpu

N, C, S = 50000, 128, 8
BN = 10000


def _route_kernel(f_ref, w_ref, b_ref, out_ref, idx_ref):
    s = pl.program_id(1)

    @pl.when(s == 0)
    def _():
        logits = jnp.dot(f_ref[...], w_ref[...], preferred_element_type=jnp.float32)
        logits = logits + b_ref[...]
        idx_ref[...] = jnp.argmax(logits, axis=1).astype(jnp.int32)[:, None]

    sel = idx_ref[...][None, :, :] == s
    out_ref[...] = jnp.where(sel, f_ref[...][None, :, :], 0.0)


@functools.partial(jax.jit, static_argnames=())
def kernel(features, W, b):
    grid = (N // BN, S)
    return pl.pallas_call(
        _route_kernel,
        grid=grid,
        in_specs=[
            pl.BlockSpec((BN, C), lambda i, s: (i, 0)),
            pl.BlockSpec((C, S), lambda i, s: (0, 0)),
            pl.BlockSpec((S,), lambda i, s: (0,)),
        ],
        out_specs=pl.BlockSpec((1, BN, C), lambda i, s: (s, i, 0)),
        out_shape=jax.ShapeDtypeStruct((S, N, C), jnp.float32),
        scratch_shapes=[pltpu.VMEM((BN, 1), jnp.int32)],
        compiler_params=pltpu.CompilerParams(
            dimension_semantics=("parallel", "arbitrary"),
        ),
    )(features, W, b)


# speedup vs baseline: 2.0069x; 1.3650x over previous
import functools

import jax
import jax.numpy as jnp
from jax.experimental import pallas as pl
from jax.experimental.pallas import tpu as pltpu

N, C, S = 50000, 128, 8
BN = 5000


def _route_kernel(f_ref, w_ref, b_ref, out_ref):
    f = f_ref[...]
    logits = jnp.dot(f, w_ref[...], preferred_element_type=jnp.float32)
    logits = logits + b_ref[...]
    idx = jnp.argmax(logits, axis=1)
    sel = idx[None, :, None] == jax.lax.broadcasted_iota(jnp.int32, (S, BN, 1), 0)
    out_ref[...] = jnp.where(sel, f[None, :, :], 0.0)


@functools.partial(jax.jit, static_argnames=())
def kernel(features, W, b):
    grid = (N // BN,)
    return pl.pallas_call(
        _route_kernel,
        grid=grid,
        in_specs=[
            pl.BlockSpec((BN, C), lambda i: (i, 0)),
            pl.BlockSpec((C, S), lambda i: (0, 0)),
            pl.BlockSpec((S,), lambda i: (0,)),
        ],
        out_specs=pl.BlockSpec((S, BN, C), lambda i: (0, i, 0)),
        out_shape=jax.ShapeDtypeStruct((S, N, C), jnp.float32),
        compiler_params=pltpu.CompilerParams(
            dimension_semantics=("parallel",),
        ),
    )(features, W, b)
